# Initial kernel scaffold; baseline (speedup 1.0000x reference)
import jax
import jax.numpy as jnp
from jax.experimental import pallas as pl


def kernel(x, ent_tables):
    raise NotImplementedError("write your pallas kernel here")


# speedup vs baseline: 2.2191x; 2.2191x over previous
import functools

import jax
import jax.numpy as jnp
from jax import lax
from jax.experimental import pallas as pl
from jax.experimental.pallas import tpu as pltpu
from jax.experimental.pallas import tpu_sc as plsc

NUM_ENTITIES = 100000
X_DIM = 16
N_TABLES = 8
BATCH = 16384

NC = 2
NS = 16
NW = NC * NS
B_PER_W = BATCH // NW
CHUNK = 128
N_CHUNKS = B_PER_W // CHUNK


@functools.partial(
    pl.kernel,
    mesh=plsc.VectorSubcoreMesh(core_axis_name="c", subcore_axis_name="s"),
    out_type=jax.ShapeDtypeStruct((N_TABLES, BATCH, X_DIM), jnp.float32),
    scratch_types=[
        pltpu.VMEM((N_CHUNKS, CHUNK), jnp.int32),
        pltpu.VMEM((N_TABLES, B_PER_W, X_DIM), jnp.float32),
        pltpu.SemaphoreType.DMA,
        pltpu.SemaphoreType.DMA,
    ],
    compiler_params=pltpu.CompilerParams(use_tc_tiling_on_sc=False),
)
def _stacked_gather(x_hbm, tab_hbm, out_hbm, idx_v, rows_v, gsem, wsem):
    wid = lax.axis_index("s") * NC + lax.axis_index("c")
    base = wid * B_PER_W
    pltpu.sync_copy(x_hbm.at[wid], idx_v)
    gathers = []
    for t in range(N_TABLES):
        for c in range(N_CHUNKS):
            gathers.append(
                pltpu.async_copy(
                    tab_hbm.at[t].at[idx_v.at[c]],
                    rows_v.at[t].at[pl.ds(c * CHUNK, CHUNK)],
                    gsem,
                )
            )
    for g in gathers:
        g.wait()
    writes = []
    for t in range(N_TABLES):
        writes.append(
            pltpu.async_copy(
                rows_v.at[t], out_hbm.at[t].at[pl.ds(base, B_PER_W)], wsem
            )
        )
    for w in writes:
        w.wait()


def kernel(x, ent_tables):
    x_r = x.reshape(NW, N_CHUNKS, CHUNK)
    return _stacked_gather(x_r, ent_tables)


# speedup vs baseline: 2.3034x; 1.0380x over previous
import functools

import jax
import jax.numpy as jnp
from jax import lax
from jax.experimental import pallas as pl
from jax.experimental.pallas import tpu as pltpu
from jax.experimental.pallas import tpu_sc as plsc

NUM_ENTITIES = 100000
X_DIM = 16
N_TABLES = 8
BATCH = 16384

NC = 2
NS = 16
NW = NC * NS
B_PER_W = BATCH // NW
CHUNK = 128
N_CHUNKS = B_PER_W // CHUNK
FB = X_DIM // 8
L = 16

WRITE_BYTES = N_CHUNKS * 8 * CHUNK * 4


@functools.partial(
    pl.kernel,
    mesh=plsc.VectorSubcoreMesh(core_axis_name="c", subcore_axis_name="s"),
    out_type=jax.ShapeDtypeStruct(
        (N_TABLES, FB, NW, N_CHUNKS, 8, CHUNK), jnp.float32
    ),
    scratch_types=[
        pltpu.VMEM((N_CHUNKS, CHUNK), jnp.int32),
        pltpu.VMEM((N_TABLES, N_CHUNKS, CHUNK, X_DIM), jnp.float32),
        pltpu.VMEM((2, FB, N_CHUNKS, 8, CHUNK), jnp.float32),
        pltpu.SemaphoreType.DMA,
        pltpu.SemaphoreType.DMA,
        pltpu.SemaphoreType.DMA,
    ],
    compiler_params=pltpu.CompilerParams(
        use_tc_tiling_on_sc=False, needs_layout_passes=False
    ),
)
def _stacked_gather(x_hbm, tab_hbm, out_hbm, idx_v, rows_v, outT_v, gsem, wsem0, wsem1):
    wid = lax.axis_index("s") * NC + lax.axis_index("c")
    base = wid * B_PER_W
    for c in range(N_CHUNKS):
        pltpu.sync_copy(x_hbm.at[pl.ds(base + c * CHUNK, CHUNK)], idx_v.at[c])
    gathers = []
    for t in range(N_TABLES):
        for c in range(N_CHUNKS):
            gathers.append(
                pltpu.async_copy(
                    tab_hbm.at[t].at[idx_v.at[c]], rows_v.at[t, c], gsem
                )
            )
    for g in gathers:
        g.wait()

    row_iotas = [jnp.arange(16, dtype=jnp.int32) + b0 for b0 in range(0, CHUNK, L)]
    col_splats = [jnp.full((L,), f, dtype=jnp.int32) for f in range(X_DIM)]

    def drain_writes(sem, par):
        for fb in range(FB):
            pltpu.make_async_copy(out_hbm.at[0, 0, 0], outT_v.at[par, fb], sem).wait()

    def table_body(t, _):
        par = t % 2

        @pl.when(t >= 2)
        def _wait_prev():
            @pl.when(par == 0)
            def _():
                drain_writes(wsem0, par)

            @pl.when(par == 1)
            def _():
                drain_writes(wsem1, par)

        for c in range(N_CHUNKS):
            for f in range(X_DIM):
                for bi in range(CHUNK // L):
                    vec = plsc.load_gather(
                        rows_v.at[t, c], [row_iotas[bi], col_splats[f]]
                    )
                    outT_v[par, f // 8, c, f % 8, pl.ds(bi * L, L)] = vec

        for fb in range(FB):
            src = outT_v.at[par, fb]

            @pl.when(par == 0)
            def _():
                pltpu.async_copy(src, out_hbm.at[t, fb, wid], wsem0)

            @pl.when(par == 1)
            def _():
                pltpu.async_copy(src, out_hbm.at[t, fb, wid], wsem1)

        return None

    lax.fori_loop(0, N_TABLES, table_body, None)
    drain_writes(wsem0, 0)
    drain_writes(wsem1, 1)


def kernel(x, ent_tables):
    raw = _stacked_gather(x, ent_tables)
    return raw.transpose(0, 2, 3, 5, 1, 4).reshape(N_TABLES, BATCH, X_DIM)


# speedup vs baseline: 12.2818x; 5.3320x over previous
import functools

import jax
import jax.numpy as jnp
from jax import lax
from jax.experimental import pallas as pl
from jax.experimental.pallas import tpu as pltpu
from jax.experimental.pallas import tpu_sc as plsc

NUM_ENTITIES = 100000
X_DIM = 16
N_TABLES = 8
BATCH = 16384

NC = 2
NS = 16
NW = NC * NS
N_ROWS = N_TABLES * X_DIM
ROWS_PER_W = N_ROWS // NW
FB = X_DIM // 8
L = 16
BB = BATCH // 128
HALF_BB = BB // 2

OUT_BLOCK_BYTES = 128 * 4


@functools.partial(
    pl.kernel,
    mesh=plsc.VectorSubcoreMesh(core_axis_name="c", subcore_axis_name="s"),
    out_type=jax.ShapeDtypeStruct(
        (N_TABLES, FB, NW, 4, 8, 128), jnp.float32
    ),
    scratch_types=[
        pltpu.VMEM((BATCH,), jnp.int32),
        pltpu.VMEM((NUM_ENTITIES,), jnp.float32),
        pltpu.VMEM((HALF_BB * 128,), jnp.float32),
        pltpu.SemaphoreType.DMA,
    ],
    compiler_params=pltpu.CompilerParams(
        use_tc_tiling_on_sc=True, needs_layout_passes=False
    ),
)
def _stacked_gather(x_hbm, tab_hbm, out_hbm, idx_v, row_v, ostage_v, wsem):
    wid = lax.axis_index("s") * NC + lax.axis_index("c")
    pltpu.sync_copy(x_hbm, idx_v)

    def drain_half():
        pltpu.make_async_copy(
            x_hbm.at[pl.ds(0, HALF_BB * 128)],
            idx_v.at[pl.ds(0, HALF_BB * 128)],
            wsem,
        ).wait()

    for k in range(ROWS_PER_W):
        r = wid * ROWS_PER_W + k
        t = r // X_DIM
        f = r % X_DIM
        fb = f // 8
        f_in = f % 8
        pltpu.sync_copy(tab_hbm.at[t, f], row_v)
        for half in range(2):
            if k > 0 or half > 0:
                drain_half()

            def bb_body(bb, _):
                bb_g = half * HALF_BB + bb
                b0 = bb_g * 128
                for j in range(8):
                    ivec = idx_v[pl.ds(b0 + j * L, L)]
                    vec = plsc.load_gather(row_v, [ivec])
                    ostage_v[pl.ds(bb * 128 + j * L, L)] = vec
                pltpu.async_copy(
                    ostage_v.at[pl.ds(bb * 128, 128)],
                    out_hbm.at[t, fb, bb_g // 4, bb_g % 4, f_in],
                    wsem,
                )
                return None

            lax.fori_loop(0, HALF_BB, bb_body, None)
    drain_half()


def kernel(x, ent_tables):
    tt = ent_tables.transpose(0, 2, 1)
    raw = _stacked_gather(x, tt)
    return raw.transpose(0, 2, 3, 5, 1, 4).reshape(N_TABLES, BATCH, X_DIM)


# speedup vs baseline: 13.6195x; 1.1089x over previous
import functools

import jax
import jax.numpy as jnp
from jax import lax
from jax.experimental import pallas as pl
from jax.experimental.pallas import tpu as pltpu
from jax.experimental.pallas import tpu_sc as plsc

NUM_ENTITIES = 100000
X_DIM = 16
N_TABLES = 8
BATCH = 16384

NC = 2
NS = 16
NW = NC * NS
N_ROWS = N_TABLES * X_DIM
ROWS_PER_W = N_ROWS // NW
FB = X_DIM // 8
L = 16
BB = BATCH // 128
HALF_BB = BB // 2

OUT_BLOCK_BYTES = 128 * 4


@functools.partial(
    pl.kernel,
    mesh=plsc.VectorSubcoreMesh(core_axis_name="c", subcore_axis_name="s"),
    out_type=jax.ShapeDtypeStruct(
        (N_TABLES, FB, NW, 4, 8, 128), jnp.float32
    ),
    scratch_types=[
        pltpu.VMEM((BATCH,), jnp.int32),
        pltpu.VMEM((NUM_ENTITIES,), jnp.float32),
        pltpu.VMEM((HALF_BB * 128,), jnp.float32),
        pltpu.SemaphoreType.DMA,
        pltpu.SemaphoreType.DMA,
    ],
    compiler_params=pltpu.CompilerParams(
        use_tc_tiling_on_sc=True, needs_layout_passes=False
    ),
)
def _stacked_gather(x_hbm, tab_hbm, out_hbm, idx_v, row_v, ostage_v, wsem, rsem):
    wid = lax.axis_index("s") * NC + lax.axis_index("c")
    idx_copy = pltpu.async_copy(x_hbm, idx_v, rsem)

    def drain_half():
        pltpu.make_async_copy(
            x_hbm.at[pl.ds(0, HALF_BB * 128)],
            idx_v.at[pl.ds(0, HALF_BB * 128)],
            wsem,
        ).wait()

    for k in range(ROWS_PER_W):
        r = wid * ROWS_PER_W + k
        t = r // X_DIM
        f = r % X_DIM
        fb = f // 8
        f_in = f % 8
        row_copy = pltpu.async_copy(tab_hbm.at[t, f], row_v, rsem)
        if k == 0:
            idx_copy.wait()
        row_copy.wait()
        for half in range(2):
            if k > 0 or half > 0:
                drain_half()

            def bb_body(bb, _):
                bb_g = half * HALF_BB + bb
                b0 = bb_g * 128
                for j in range(8):
                    ivec = idx_v[pl.ds(b0 + j * L, L)]
                    vec = plsc.load_gather(row_v, [ivec])
                    ostage_v[pl.ds(bb * 128 + j * L, L)] = vec
                pltpu.async_copy(
                    ostage_v.at[pl.ds(bb * 128, 128)],
                    out_hbm.at[t, fb, bb_g // 4, bb_g % 4, f_in],
                    wsem,
                )
                return None

            lax.fori_loop(0, HALF_BB, bb_body, None, unroll=4)
    drain_half()


def kernel(x, ent_tables):
    tt = ent_tables.transpose(0, 2, 1)
    raw = _stacked_gather(x, tt)
    return raw.transpose(0, 2, 3, 5, 1, 4).reshape(N_TABLES, BATCH, X_DIM)
